# baseline (speedup 1.0000x reference)
import functools

import jax
import jax.numpy as jnp
from jax import lax
from jax.experimental import pallas as pl
from jax.experimental.pallas import tpu as pltpu
from jax.experimental.pallas import tpu_sc as plsc

N_ROWS = 100000
LD = 16
B = 16384

_info = plsc.get_sparse_core_info()
NC = _info.num_cores
NS = _info.num_subcores
NW = NC * NS
B_PER_W = B // NW
CHUNK = 128
NCHUNK = B_PER_W // CHUNK


def _body(idx_hbm, q_mu_hbm, q_ls_hbm, eps_hbm, out_hbm,
          idx_v, mu_v, ls_v, eps_v, out_v, sem):
    wid = lax.axis_index("s") * NC + lax.axis_index("c")
    base = wid * B_PER_W
    pltpu.sync_copy(idx_hbm.at[pl.ds(wid * NCHUNK, NCHUNK)], idx_v)
    eps_cp = pltpu.async_copy(eps_hbm.at[pl.ds(base, B_PER_W)], eps_v, sem)
    cps = []
    for j in range(NCHUNK):
        sl = pl.ds(j * CHUNK, CHUNK)
        cps.append(pltpu.async_copy(q_mu_hbm.at[idx_v.at[j]], mu_v.at[sl], sem))
        cps.append(pltpu.async_copy(q_ls_hbm.at[idx_v.at[j]], ls_v.at[sl], sem))
    eps_cp.wait()
    for cp in cps:
        cp.wait()

    def row(i, _):
        out_v[i] = mu_v[i] + jnp.exp(ls_v[i]) * eps_v[i]
        return 0

    lax.fori_loop(0, B_PER_W, row, 0)
    pltpu.sync_copy(out_v, out_hbm.at[pl.ds(base, B_PER_W)])


@jax.jit
def _run(idx2d, q_mu, q_log_sigma, eps):
    mesh = plsc.VectorSubcoreMesh(core_axis_name="c", subcore_axis_name="s")
    f = functools.partial(
        pl.kernel,
        mesh=mesh,
        out_type=jax.ShapeDtypeStruct((B, LD), jnp.float32),
        scratch_types=[
            pltpu.VMEM((NCHUNK, CHUNK), jnp.int32),
            pltpu.VMEM((B_PER_W, LD), jnp.float32),
            pltpu.VMEM((B_PER_W, LD), jnp.float32),
            pltpu.VMEM((B_PER_W, LD), jnp.float32),
            pltpu.VMEM((B_PER_W, LD), jnp.float32),
            pltpu.SemaphoreType.DMA,
        ],
        compiler_params=pltpu.CompilerParams(use_tc_tiling_on_sc=False),
    )(_body)
    return f(idx2d, q_mu, q_log_sigma, eps)


def kernel(batch_idx, q_mu, q_log_sigma, prior_loc, prior_var, eps):
    del prior_loc, prior_var
    idx2d = batch_idx.astype(jnp.int32).reshape(NW * NCHUNK, CHUNK)
    return _run(idx2d, q_mu, q_log_sigma, eps)
